# Initial kernel scaffold; baseline (speedup 1.0000x reference)
import jax
import jax.numpy as jnp
from jax.experimental import pallas as pl


def kernel(xe, Lu, Ld, har_base, har_spline, sol_base, sol_spline, irr_base, irr_spline, grid):
    raise NotImplementedError("write your pallas kernel here")


# speedup vs baseline: 17.1429x; 17.1429x over previous
import jax
import jax.numpy as jnp
from jax import lax
from jax.experimental import pallas as pl
from jax.experimental.pallas import tpu as pltpu
from jax.experimental.pallas import tpu_sc as plsc

N = 10000
E = 320000
F = 128
NCOEF = 8
GP = 12
NTILES = 16
EPT = E // NTILES
CHUNK = 128
NFULL = EPT // CHUNK
TAIL = EPT - NFULL * CHUNK
RPT = 1000
ZROWS = 200
ROWB = 400
NBLK = N // ROWB


def _deg_body(dstu_hbm, dstd_hbm, degu_hbm, degd_hbm,
              idx_v, idxt_v, ones_v, zb_v, deg_sh):
    c = lax.axis_index("c")
    s = lax.axis_index("s")

    def fill_ones(i, carry):
        ones_v[pl.ds(i * 16, 16)] = jnp.ones((16,), jnp.float32)
        return carry
    lax.fori_loop(0, CHUNK // 16, fill_ones, 0)

    def fill_zeros(i, carry):
        zb_v[pl.ds(i * 16, 16)] = jnp.zeros((16,), jnp.float32)
        return carry
    lax.fori_loop(0, 64, fill_zeros, 0)

    @pl.when(s < 10)
    def _():
        pltpu.sync_copy(zb_v.at[pl.ds(0, 1000)], deg_sh.at[pl.ds(s * 1000, 1000)])
    plsc.subcore_barrier()

    def count(dst_hbm):
        base = s * EPT

        def chunk(ci, carry):
            off = base + ci * CHUNK
            pltpu.sync_copy(dst_hbm.at[pl.ds(off, CHUNK)], idx_v)
            pltpu.sync_copy(ones_v, deg_sh.at[idx_v], add=True)
            return carry
        lax.fori_loop(0, NFULL, chunk, 0)
        off = base + NFULL * CHUNK
        pltpu.sync_copy(dst_hbm.at[pl.ds(off, TAIL)], idxt_v)
        pltpu.sync_copy(ones_v.at[pl.ds(0, TAIL)], deg_sh.at[idxt_v], add=True)

    @pl.when(c == 0)
    def _():
        count(dstu_hbm)

    @pl.when(c == 1)
    def _():
        count(dstd_hbm)

    plsc.subcore_barrier()

    @pl.when(s < 10)
    def _():
        pltpu.sync_copy(deg_sh.at[pl.ds(s * 1000, 1000)], zb_v.at[pl.ds(0, 1000)])

        @pl.when(c == 0)
        def _():
            pltpu.sync_copy(zb_v.at[pl.ds(0, 1000)], degu_hbm.at[pl.ds(s * 1000, 1000)])

        @pl.when(c == 1)
        def _():
            pltpu.sync_copy(zb_v.at[pl.ds(0, 1000)], degd_hbm.at[pl.ds(s * 1000, 1000)])


def _sct_body(ys_hbm, yi_hbm, srcu_hbm, dstu_hbm, srcd_hbm, dstd_hbm,
              aggs_hbm, aggi_hbm,
              sidx_v, didx_v, sidxt_v, didxt_v, rows_v, rowst_v, zb_v, agg_sh,
              sem):
    c = lax.axis_index("c")
    s = lax.axis_index("s")

    def fill_zeros(i, carry):
        r = i // 8
        k = i - r * 8
        zb_v[r, pl.ds(k * 16, 16)] = jnp.zeros((16,), jnp.float32)
        return carry
    lax.fori_loop(0, ZROWS * 8, fill_zeros, 0)

    @pl.when(s < 10)
    def _():
        def zero_copy(q, carry):
            pltpu.sync_copy(zb_v, agg_sh.at[pl.ds(s * RPT + q * ZROWS, ZROWS)])
            return carry
        lax.fori_loop(0, RPT // ZROWS, zero_copy, 0)
    plsc.subcore_barrier()

    def conv(src_hbm, dst_hbm, y_hbm):
        base = s * EPT

        def chunk(ci, carry):
            off = base + ci * CHUNK
            pltpu.sync_copy(src_hbm.at[pl.ds(off, CHUNK)], sidx_v)
            pltpu.sync_copy(dst_hbm.at[pl.ds(off, CHUNK)], didx_v)
            pltpu.async_copy(y_hbm.at[sidx_v], rows_v, sem).wait()
            pltpu.sync_copy(rows_v, agg_sh.at[didx_v], add=True)
            return carry
        lax.fori_loop(0, NFULL, chunk, 0)
        off = base + NFULL * CHUNK
        pltpu.sync_copy(src_hbm.at[pl.ds(off, TAIL)], sidxt_v)
        pltpu.sync_copy(dst_hbm.at[pl.ds(off, TAIL)], didxt_v)
        pltpu.async_copy(y_hbm.at[sidxt_v], rowst_v, sem).wait()
        pltpu.sync_copy(rowst_v, agg_sh.at[didxt_v], add=True)

    @pl.when(c == 0)
    def _():
        conv(srcu_hbm, dstu_hbm, ys_hbm)

    @pl.when(c == 1)
    def _():
        conv(srcd_hbm, dstd_hbm, yi_hbm)

    plsc.subcore_barrier()

    def copy_out(out_hbm):
        def piece(q, carry):
            off = s * RPT + q * ZROWS
            pltpu.sync_copy(agg_sh.at[pl.ds(off, ZROWS)], zb_v)
            pltpu.sync_copy(zb_v, out_hbm.at[pl.ds(off, ZROWS)])
            return carry
        lax.fori_loop(0, RPT // ZROWS, piece, 0)

    @pl.when(s < 10)
    def _():
        @pl.when(c == 0)
        def _():
            copy_out(aggs_hbm)

        @pl.when(c == 1)
        def _():
            copy_out(aggi_hbm)


def _kan_body(x_ref, gp_ref, w_ref, du_ref, dd_ref, zh_ref, ys_ref, yi_ref):
    x = x_ref[...]
    g = [gp_ref[j] for j in range(GP)]
    b = [((x >= g[j][None, :]) & (x < g[j + 1][None, :])).astype(jnp.float32)
         for j in range(GP - 1)]
    for p in range(1, 4):
        nb = []
        for j in range(GP - 1 - p):
            r1 = (1.0 / (g[j + p] - g[j]))[None, :]
            r2 = (1.0 / (g[j + p + 1] - g[j + 1]))[None, :]
            nb.append((x - g[j][None, :]) * r1 * b[j]
                      + (g[j + p + 1][None, :] - x) * r2 * b[j + 1])
        b = nb
    feat = jnp.concatenate([x * jax.nn.sigmoid(x)] + b, axis=1)
    z = jnp.dot(feat, w_ref[...], preferred_element_type=jnp.float32)
    du = du_ref[...]
    dd = dd_ref[...]
    disu = jnp.where(du > 0.0, lax.rsqrt(du), 0.0)
    disd = jnp.where(dd > 0.0, lax.rsqrt(dd), 0.0)
    zh_ref[...] = z[:, :F]
    ys_ref[...] = z[:, F:2 * F] * disu
    yi_ref[...] = z[:, 2 * F:3 * F] * disd


def _fin_body(zh_ref, as_ref, ai_ref, du_ref, dd_ref, o_ref):
    du = du_ref[...]
    dd = dd_ref[...]
    disu = jnp.where(du > 0.0, lax.rsqrt(du), 0.0)
    disd = jnp.where(dd > 0.0, lax.rsqrt(dd), 0.0)
    o_ref[...] = zh_ref[...] + as_ref[...] * disu + ai_ref[...] * disd


def kernel(xe, Lu, Ld, har_base, har_spline, sol_base, sol_spline,
           irr_base, irr_spline, grid):
    f32 = jnp.float32
    srcu = Lu[0].astype(jnp.int32)
    dstu = Lu[1].astype(jnp.int32)
    srcd = Ld[0].astype(jnp.int32)
    dstd = Ld[1].astype(jnp.int32)

    base_cat = jnp.concatenate([har_base, sol_base, irr_base], axis=0)
    spl_cat = jnp.concatenate([har_spline, sol_spline, irr_spline], axis=0)
    w = jnp.concatenate(
        [base_cat.T, jnp.transpose(spl_cat, (2, 1, 0)).reshape(NCOEF * F, 3 * F)],
        axis=0)
    gp = grid.astype(f32).T

    mesh = plsc.VectorSubcoreMesh(core_axis_name="c", subcore_axis_name="s")
    deg_call = pl.kernel(
        _deg_body,
        out_type=(jax.ShapeDtypeStruct((N,), f32),
                  jax.ShapeDtypeStruct((N,), f32)),
        mesh=mesh,
        scratch_types=[
            pltpu.VMEM((CHUNK,), jnp.int32),
            pltpu.VMEM((TAIL,), jnp.int32),
            pltpu.VMEM((CHUNK,), f32),
            pltpu.VMEM((1024,), f32),
            pltpu.VMEM_SHARED((N,), f32),
        ],
    )
    degu, degd = deg_call(dstu, dstd)
    du2 = degu.reshape(N, 1)
    dd2 = degd.reshape(N, 1)

    zh, ys, yi = pl.pallas_call(
        _kan_body,
        grid=(NBLK,),
        in_specs=[
            pl.BlockSpec((ROWB, F), lambda i: (i, 0)),
            pl.BlockSpec((GP, F), lambda i: (0, 0)),
            pl.BlockSpec(((1 + NCOEF) * F, 3 * F), lambda i: (0, 0)),
            pl.BlockSpec((ROWB, 1), lambda i: (i, 0)),
            pl.BlockSpec((ROWB, 1), lambda i: (i, 0)),
        ],
        out_specs=[pl.BlockSpec((ROWB, F), lambda i: (i, 0))] * 3,
        out_shape=[jax.ShapeDtypeStruct((N, F), f32)] * 3,
    )(xe, gp, w, du2, dd2)

    sct_call = pl.kernel(
        _sct_body,
        out_type=(jax.ShapeDtypeStruct((N, F), f32),
                  jax.ShapeDtypeStruct((N, F), f32)),
        mesh=mesh,
        scratch_types=[
            pltpu.VMEM((CHUNK,), jnp.int32),
            pltpu.VMEM((CHUNK,), jnp.int32),
            pltpu.VMEM((TAIL,), jnp.int32),
            pltpu.VMEM((TAIL,), jnp.int32),
            pltpu.VMEM((CHUNK, F), f32),
            pltpu.VMEM((TAIL, F), f32),
            pltpu.VMEM((ZROWS, F), f32),
            pltpu.VMEM_SHARED((N, F), f32),
            pltpu.SemaphoreType.DMA,
        ],
    )
    aggs, aggi = sct_call(ys, yi, srcu, dstu, srcd, dstd)

    out = pl.pallas_call(
        _fin_body,
        grid=(NBLK,),
        in_specs=[
            pl.BlockSpec((ROWB, F), lambda i: (i, 0)),
            pl.BlockSpec((ROWB, F), lambda i: (i, 0)),
            pl.BlockSpec((ROWB, F), lambda i: (i, 0)),
            pl.BlockSpec((ROWB, 1), lambda i: (i, 0)),
            pl.BlockSpec((ROWB, 1), lambda i: (i, 0)),
        ],
        out_specs=pl.BlockSpec((ROWB, F), lambda i: (i, 0)),
        out_shape=jax.ShapeDtypeStruct((N, F), f32),
    )(zh, aggs, aggi, du2, dd2)
    return out


# speedup vs baseline: 18.2107x; 1.0623x over previous
import jax
import jax.numpy as jnp
from jax import lax
from jax.experimental import pallas as pl
from jax.experimental.pallas import tpu as pltpu
from jax.experimental.pallas import tpu_sc as plsc

N = 10000
E = 320000
F = 128
NCOEF = 8
GP = 12
NTILES = 16
EPT = E // NTILES
CHUNK = 128
NCH = 160
IDXB = 40
NSTG = NCH // IDXB
NPAD = N + 16
RPT = 1000
ROWB = 400
NBLK = N // ROWB


def _deg_body(dstu_hbm, dstd_hbm, degu_hbm, degd_hbm,
              didx_v, ones_v, zb_v, deg_sh, sem):
    c = lax.axis_index("c")
    s = lax.axis_index("s")

    def fill_ones(i, carry):
        ones_v[pl.ds(i * 16, 16)] = jnp.ones((16,), jnp.float32)
        return carry
    lax.fori_loop(0, CHUNK // 16, fill_ones, 0)

    def fill_zeros(i, carry):
        zb_v[pl.ds(i * 16, 16)] = jnp.zeros((16,), jnp.float32)
        return carry
    lax.fori_loop(0, 64, fill_zeros, 0)

    @pl.when(s < 10)
    def _():
        pltpu.sync_copy(zb_v.at[pl.ds(0, 1000)], deg_sh.at[pl.ds(s * 1000, 1000)])
    plsc.subcore_barrier()

    @pl.when(c == 0)
    def _():
        pltpu.sync_copy(dstu_hbm.at[s], didx_v)

    @pl.when(c == 1)
    def _():
        pltpu.sync_copy(dstd_hbm.at[s], didx_v)

    def fire(j):
        pltpu.async_copy(ones_v, deg_sh.at[didx_v.at[j]], sem, add=True)

    def drain(j):
        pltpu.make_async_copy(ones_v, deg_sh.at[didx_v.at[j]], sem).wait()

    for j in range(4):
        fire(j)

    def step(j, carry):
        drain(j)

        @pl.when(j + 4 < NCH)
        def _():
            fire(j + 4)
        return carry
    lax.fori_loop(0, NCH, step, 0)

    plsc.subcore_barrier()

    @pl.when(s < 10)
    def _():
        pltpu.sync_copy(deg_sh.at[pl.ds(s * 1000, 1000)], zb_v.at[pl.ds(0, 1000)])

        @pl.when(c == 0)
        def _():
            pltpu.sync_copy(zb_v.at[pl.ds(0, 1000)], degu_hbm.at[pl.ds(s * 1000, 1000)])

        @pl.when(c == 1)
        def _():
            pltpu.sync_copy(zb_v.at[pl.ds(0, 1000)], degd_hbm.at[pl.ds(s * 1000, 1000)])


def _sct_body(ys_hbm, yi_hbm, srcu_hbm, dstu_hbm, srcd_hbm, dstd_hbm,
              aggs_hbm, aggi_hbm,
              sidx_v, didx_v, ra, rb, agg_sh, gsa, gsb):
    c = lax.axis_index("c")
    s = lax.axis_index("s")

    def fill_zeros(i, carry):
        r = i // 8
        k = i - r * 8
        ra[r, pl.ds(k * 16, 16)] = jnp.zeros((16,), jnp.float32)
        return carry
    lax.fori_loop(0, CHUNK * 8, fill_zeros, 0)

    @pl.when(s < 10)
    def _():
        def zero_copy(q, carry):
            pltpu.sync_copy(ra, agg_sh.at[pl.ds(s * RPT + q * CHUNK, CHUNK)])
            return carry
        lax.fori_loop(0, 7, zero_copy, 0)
        pltpu.sync_copy(ra.at[pl.ds(0, 104)], agg_sh.at[pl.ds(s * RPT + 896, 104)])

    plsc.subcore_barrier()

    def conv(y_hbm, src_hbm, dst_hbm):
        def stage(st, carry):
            pltpu.sync_copy(src_hbm.at[s, pl.ds(st * IDXB, IDXB)], sidx_v)
            pltpu.sync_copy(dst_hbm.at[s, pl.ds(st * IDXB, IDXB)], didx_v)
            pltpu.async_copy(y_hbm.at[sidx_v.at[0]], ra, gsa)

            def pair(p, carry2):
                j = 2 * p
                pltpu.async_copy(y_hbm.at[sidx_v.at[j + 1]], rb, gsb)
                pltpu.make_async_copy(y_hbm.at[sidx_v.at[j]], ra, gsa).wait()
                pltpu.sync_copy(ra, agg_sh.at[didx_v.at[j]], add=True)

                @pl.when(j + 2 < IDXB)
                def _():
                    pltpu.async_copy(y_hbm.at[sidx_v.at[j + 2]], ra, gsa)
                pltpu.make_async_copy(y_hbm.at[sidx_v.at[j + 1]], rb, gsb).wait()
                pltpu.sync_copy(rb, agg_sh.at[didx_v.at[j + 1]], add=True)
                return carry2
            lax.fori_loop(0, IDXB // 2, pair, 0)
            return carry
        lax.fori_loop(0, NSTG, stage, 0)

    @pl.when(c == 0)
    def _():
        conv(ys_hbm, srcu_hbm, dstu_hbm)

    @pl.when(c == 1)
    def _():
        conv(yi_hbm, srcd_hbm, dstd_hbm)

    plsc.subcore_barrier()

    def copy_out(out_hbm):
        def piece(q, carry):
            off = s * RPT + q * CHUNK
            pltpu.sync_copy(agg_sh.at[pl.ds(off, CHUNK)], ra)
            pltpu.sync_copy(ra, out_hbm.at[pl.ds(off, CHUNK)])
            return carry
        lax.fori_loop(0, 7, piece, 0)
        off = s * RPT + 896
        pltpu.sync_copy(agg_sh.at[pl.ds(off, 104)], ra.at[pl.ds(0, 104)])
        pltpu.sync_copy(ra.at[pl.ds(0, 104)], out_hbm.at[pl.ds(off, 104)])

    @pl.when(s < 10)
    def _():
        @pl.when(c == 0)
        def _():
            copy_out(aggs_hbm)

        @pl.when(c == 1)
        def _():
            copy_out(aggi_hbm)


def _kan_body(x_ref, gp_ref, w_ref, du_ref, dd_ref, zh_ref, ys_ref, yi_ref):
    x = x_ref[...]
    g = [gp_ref[j] for j in range(GP)]
    b = [((x >= g[j][None, :]) & (x < g[j + 1][None, :])).astype(jnp.float32)
         for j in range(GP - 1)]
    for p in range(1, 4):
        nb = []
        for j in range(GP - 1 - p):
            r1 = (1.0 / (g[j + p] - g[j]))[None, :]
            r2 = (1.0 / (g[j + p + 1] - g[j + 1]))[None, :]
            nb.append((x - g[j][None, :]) * r1 * b[j]
                      + (g[j + p + 1][None, :] - x) * r2 * b[j + 1])
        b = nb
    feat = jnp.concatenate([x * jax.nn.sigmoid(x)] + b, axis=1)
    z = jnp.dot(feat, w_ref[...], preferred_element_type=jnp.float32)
    du = du_ref[...]
    dd = dd_ref[...]
    disu = jnp.where(du > 0.0, lax.rsqrt(du), 0.0)
    disd = jnp.where(dd > 0.0, lax.rsqrt(dd), 0.0)
    zh_ref[...] = z[:, :F]
    ys_ref[...] = z[:, F:2 * F] * disu
    yi_ref[...] = z[:, 2 * F:3 * F] * disd


def _fin_body(zh_ref, as_ref, ai_ref, du_ref, dd_ref, o_ref):
    du = du_ref[...]
    dd = dd_ref[...]
    disu = jnp.where(du > 0.0, lax.rsqrt(du), 0.0)
    disd = jnp.where(dd > 0.0, lax.rsqrt(dd), 0.0)
    o_ref[...] = zh_ref[...] + as_ref[...] * disu + ai_ref[...] * disd


def kernel(xe, Lu, Ld, har_base, har_spline, sol_base, sol_spline,
           irr_base, irr_spline, grid):
    f32 = jnp.float32

    def pad_idx(v, pad_val):
        v2 = v.astype(jnp.int32).reshape(NTILES, EPT)
        v2 = jnp.pad(v2, ((0, 0), (0, NCH * CHUNK - EPT)), constant_values=pad_val)
        return v2.reshape(NTILES, NCH, CHUNK)

    srcu = pad_idx(Lu[0], 0)
    dstu = pad_idx(Lu[1], N)
    srcd = pad_idx(Ld[0], 0)
    dstd = pad_idx(Ld[1], N)

    base_cat = jnp.concatenate([har_base, sol_base, irr_base], axis=0)
    spl_cat = jnp.concatenate([har_spline, sol_spline, irr_spline], axis=0)
    w = jnp.concatenate(
        [base_cat.T, jnp.transpose(spl_cat, (2, 1, 0)).reshape(NCOEF * F, 3 * F)],
        axis=0)
    gp = grid.astype(f32).T

    mesh = plsc.VectorSubcoreMesh(core_axis_name="c", subcore_axis_name="s")
    deg_call = pl.kernel(
        _deg_body,
        out_type=(jax.ShapeDtypeStruct((N,), f32),
                  jax.ShapeDtypeStruct((N,), f32)),
        mesh=mesh,
        scratch_types=[
            pltpu.VMEM((NCH, CHUNK), jnp.int32),
            pltpu.VMEM((CHUNK,), f32),
            pltpu.VMEM((1024,), f32),
            pltpu.VMEM_SHARED((NPAD,), f32),
            pltpu.SemaphoreType.DMA,
        ],
    )
    degu, degd = deg_call(dstu, dstd)
    du2 = degu.reshape(N, 1)
    dd2 = degd.reshape(N, 1)

    zh, ys, yi = pl.pallas_call(
        _kan_body,
        grid=(NBLK,),
        in_specs=[
            pl.BlockSpec((ROWB, F), lambda i: (i, 0)),
            pl.BlockSpec((GP, F), lambda i: (0, 0)),
            pl.BlockSpec(((1 + NCOEF) * F, 3 * F), lambda i: (0, 0)),
            pl.BlockSpec((ROWB, 1), lambda i: (i, 0)),
            pl.BlockSpec((ROWB, 1), lambda i: (i, 0)),
        ],
        out_specs=[pl.BlockSpec((ROWB, F), lambda i: (i, 0))] * 3,
        out_shape=[jax.ShapeDtypeStruct((N, F), f32)] * 3,
    )(xe, gp, w, du2, dd2)

    sct_call = pl.kernel(
        _sct_body,
        out_type=(jax.ShapeDtypeStruct((N, F), f32),
                  jax.ShapeDtypeStruct((N, F), f32)),
        mesh=mesh,
        scratch_types=[
            pltpu.VMEM((IDXB, CHUNK), jnp.int32),
            pltpu.VMEM((IDXB, CHUNK), jnp.int32),
            pltpu.VMEM((CHUNK, F), f32),
            pltpu.VMEM((CHUNK, F), f32),
            pltpu.VMEM_SHARED((NPAD, F), f32),
            pltpu.SemaphoreType.DMA,
            pltpu.SemaphoreType.DMA,
        ],
    )
    aggs, aggi = sct_call(ys, yi, srcu, dstu, srcd, dstd)

    out = pl.pallas_call(
        _fin_body,
        grid=(NBLK,),
        in_specs=[
            pl.BlockSpec((ROWB, F), lambda i: (i, 0)),
            pl.BlockSpec((ROWB, F), lambda i: (i, 0)),
            pl.BlockSpec((ROWB, F), lambda i: (i, 0)),
            pl.BlockSpec((ROWB, 1), lambda i: (i, 0)),
            pl.BlockSpec((ROWB, 1), lambda i: (i, 0)),
        ],
        out_specs=pl.BlockSpec((ROWB, F), lambda i: (i, 0)),
        out_shape=jax.ShapeDtypeStruct((N, F), f32),
    )(zh, aggs, aggi, du2, dd2)
    return out
